# baseline (speedup 1.0000x reference)
import functools

import jax
import jax.numpy as jnp
from jax import lax
from jax.experimental import pallas as pl
from jax.experimental.pallas import tpu as pltpu
from jax.experimental.pallas import tpu_sc as plsc

_B, _S, _H = 4, 2048, 768
_R = _B * _S
_NC, _NS, _L = 2, 16, 16
_NW = _NC * _NS
_RPW = _R // _NW
_CH = 64
_NCHUNK = _RPW // _CH

_BLK = 256


def _layer2_body(x_ref, h_ref, m_ref, ws_ref, wh_ref, b_ref, o_ref, os_ref):
    x = x_ref[...]
    h = h_ref[...]
    acc = jnp.dot(x, ws_ref[...], preferred_element_type=jnp.float32)
    acc += jnp.dot(h, wh_ref[...], preferred_element_type=jnp.float32)
    o = jnp.maximum(acc + b_ref[...], 0.0) * m_ref[...]
    o_ref[...] = o
    os_ref[...] = o


def _layer_body(x_ref, h_ref, m_ref, ws_ref, wh_ref, b_ref, o_ref):
    x = x_ref[...]
    h = h_ref[...]
    acc = jnp.dot(x, ws_ref[...], preferred_element_type=jnp.float32)
    acc += jnp.dot(h, wh_ref[...], preferred_element_type=jnp.float32)
    o_ref[...] = jnp.maximum(acc + b_ref[...], 0.0) * m_ref[...]


_row_spec = pl.BlockSpec((_BLK, _H), lambda i: (i, 0))
_mask_spec = pl.BlockSpec((_BLK, 1), lambda i: (i, 0))
_w_spec = pl.BlockSpec((_H, _H), lambda i: (0, 0))
_b_spec = pl.BlockSpec((1, _H), lambda i: (0, 0))

_tc_layer1 = pl.pallas_call(
    _layer2_body,
    grid=(_R // _BLK,),
    in_specs=[_row_spec, _row_spec, _mask_spec, _w_spec, _w_spec, _b_spec],
    out_specs=(_row_spec, _row_spec),
    out_shape=(jax.ShapeDtypeStruct((_R, _H), jnp.float32),
               jax.ShapeDtypeStruct((_R, _H), jnp.float32)),
)

_tc_layer2 = pl.pallas_call(
    _layer_body,
    grid=(_R // _BLK,),
    in_specs=[_row_spec, _row_spec, _mask_spec, _w_spec, _w_spec, _b_spec],
    out_specs=_row_spec,
    out_shape=jax.ShapeDtypeStruct((_R, _H), jnp.float32),
)


def _sc_gather_body(heads_hbm, table_hbm, out_hbm, idx_v, buf0, buf1, sem0, sem1):
    wid = lax.axis_index("s") * _NC + lax.axis_index("c")
    base = wid * _RPW
    pltpu.sync_copy(heads_hbm.at[pl.ds(base, _RPW)], idx_v)
    off = (base // _S) * _S
    for j in range(_RPW // _L):
        sl = pl.ds(j * _L, _L)
        idx_v[sl] = idx_v[sl] + off
    bufs, sems = (buf0, buf1), (sem0, sem1)
    cps = []
    for i in range(_NCHUNK):
        cp = pltpu.make_async_copy(
            table_hbm.at[idx_v.at[pl.ds(i * _CH, _CH)]], bufs[i % 2], sems[i % 2])
        cp.start()
        cps.append(cp)
        if i >= 1:
            cps[i - 1].wait()
            pltpu.sync_copy(bufs[(i - 1) % 2],
                            out_hbm.at[pl.ds(base + (i - 1) * _CH, _CH)])
    cps[-1].wait()
    pltpu.sync_copy(bufs[(_NCHUNK - 1) % 2],
                    out_hbm.at[pl.ds(base + (_NCHUNK - 1) * _CH, _CH)])


@functools.cache
def _make_sc_gather():
    return pl.kernel(
        _sc_gather_body,
        out_type=jax.ShapeDtypeStruct((_R, _H), jnp.float32),
        mesh=plsc.VectorSubcoreMesh(core_axis_name="c", subcore_axis_name="s"),
        scratch_types=[
            pltpu.VMEM((_RPW,), jnp.int32),
            pltpu.VMEM((_CH, _H), jnp.float32),
            pltpu.VMEM((_CH, _H), jnp.float32),
            pltpu.SemaphoreType.DMA,
            pltpu.SemaphoreType.DMA,
        ],
    )


def kernel(hidden_states, attention_mask, heads, rels, W_self, W_head, b):
    del rels
    x0 = hidden_states.reshape(_R, _H)
    mask = attention_mask.reshape(_R, 1)
    hflat = heads.reshape(_R).astype(jnp.int32)

    sc_gather = _make_sc_gather()
    h1 = sc_gather(hflat, x0)
    x1, x1s = _tc_layer1(x0, h1, mask, W_self[0], W_head[0], b[0].reshape(1, _H))
    h2 = sc_gather(hflat, x1s)
    x2 = _tc_layer2(x1, h2, mask, W_self[1], W_head[1], b[1].reshape(1, _H))
    return x2.reshape(_B, _S, _H)


# speedup vs baseline: 1.1378x; 1.1378x over previous
import functools

import jax
import jax.numpy as jnp
from jax import lax
from jax.experimental import pallas as pl
from jax.experimental.pallas import tpu as pltpu
from jax.experimental.pallas import tpu_sc as plsc

_B, _S, _H = 4, 2048, 768
_R = _B * _S
_NC, _NS, _L = 2, 16, 16
_NW = _NC * _NS
_RPW = _R // _NW
_CH = 64
_NCHUNK = _RPW // _CH

_BLK = 512


def _layer2_body(x_ref, h_ref, m_ref, ws_ref, wh_ref, b_ref, o_ref, os_ref):
    x = x_ref[...]
    h = h_ref[...]
    acc = jnp.dot(x, ws_ref[...], preferred_element_type=jnp.float32)
    acc += jnp.dot(h, wh_ref[...], preferred_element_type=jnp.float32)
    o = jnp.maximum(acc + b_ref[...], 0.0) * m_ref[...]
    o_ref[...] = o
    os_ref[...] = o


def _layer_body(x_ref, h_ref, m_ref, ws_ref, wh_ref, b_ref, o_ref):
    x = x_ref[...]
    h = h_ref[...]
    acc = jnp.dot(x, ws_ref[...], preferred_element_type=jnp.float32)
    acc += jnp.dot(h, wh_ref[...], preferred_element_type=jnp.float32)
    o_ref[...] = jnp.maximum(acc + b_ref[...], 0.0) * m_ref[...]


_row_spec = pl.BlockSpec((_BLK, _H), lambda i: (i, 0))
_mask_spec = pl.BlockSpec((_BLK, 1), lambda i: (i, 0))
_w_spec = pl.BlockSpec((_H, _H), lambda i: (0, 0))
_b_spec = pl.BlockSpec((1, _H), lambda i: (0, 0))

_tc_layer1 = pl.pallas_call(
    _layer2_body,
    grid=(_R // _BLK,),
    in_specs=[_row_spec, _row_spec, _mask_spec, _w_spec, _w_spec, _b_spec],
    out_specs=(_row_spec, _row_spec),
    out_shape=(jax.ShapeDtypeStruct((_R, _H), jnp.float32),
               jax.ShapeDtypeStruct((_R, _H), jnp.float32)),
)

_tc_layer2 = pl.pallas_call(
    _layer_body,
    grid=(_R // _BLK,),
    in_specs=[_row_spec, _row_spec, _mask_spec, _w_spec, _w_spec, _b_spec],
    out_specs=_row_spec,
    out_shape=jax.ShapeDtypeStruct((_R, _H), jnp.float32),
)


def _sc_gather_body(heads_hbm, table_hbm, out_hbm, idx_v, buf0, buf1, sem0, sem1):
    wid = lax.axis_index("s") * _NC + lax.axis_index("c")
    base = wid * _RPW
    pltpu.sync_copy(heads_hbm.at[pl.ds(base, _RPW)], idx_v)
    off = (base // _S) * _S
    for j in range(_RPW // _L):
        sl = pl.ds(j * _L, _L)
        idx_v[sl] = idx_v[sl] + off
    bufs, sems = (buf0, buf1), (sem0, sem1)
    cps = []
    for i in range(_NCHUNK):
        cp = pltpu.make_async_copy(
            table_hbm.at[idx_v.at[pl.ds(i * _CH, _CH)]], bufs[i % 2], sems[i % 2])
        cp.start()
        cps.append(cp)
        if i >= 1:
            cps[i - 1].wait()
            pltpu.sync_copy(bufs[(i - 1) % 2],
                            out_hbm.at[pl.ds(base + (i - 1) * _CH, _CH)])
    cps[-1].wait()
    pltpu.sync_copy(bufs[(_NCHUNK - 1) % 2],
                    out_hbm.at[pl.ds(base + (_NCHUNK - 1) * _CH, _CH)])


@functools.cache
def _make_sc_gather():
    return pl.kernel(
        _sc_gather_body,
        out_type=jax.ShapeDtypeStruct((_R, _H), jnp.float32),
        mesh=plsc.VectorSubcoreMesh(core_axis_name="c", subcore_axis_name="s"),
        scratch_types=[
            pltpu.VMEM((_RPW,), jnp.int32),
            pltpu.VMEM((_CH, _H), jnp.float32),
            pltpu.VMEM((_CH, _H), jnp.float32),
            pltpu.SemaphoreType.DMA,
            pltpu.SemaphoreType.DMA,
        ],
    )


def kernel(hidden_states, attention_mask, heads, rels, W_self, W_head, b):
    del rels
    x0 = hidden_states.reshape(_R, _H)
    mask = attention_mask.reshape(_R, 1)
    hflat = heads.reshape(_R).astype(jnp.int32)

    sc_gather = _make_sc_gather()
    h1 = sc_gather(hflat, x0)
    x1, x1s = _tc_layer1(x0, h1, mask, W_self[0], W_head[0], b[0].reshape(1, _H))
    h2 = sc_gather(hflat, x1s)
    x2 = _tc_layer2(x1, h2, mask, W_self[1], W_head[1], b[1].reshape(1, _H))
    return x2.reshape(_B, _S, _H)


# speedup vs baseline: 1.1968x; 1.0518x over previous
import functools

import jax
import jax.numpy as jnp
from jax import lax
from jax.experimental import pallas as pl
from jax.experimental.pallas import tpu as pltpu
from jax.experimental.pallas import tpu_sc as plsc

_B, _S, _H = 4, 2048, 768
_R = _B * _S
_NC, _NS, _L = 2, 16, 16
_NW = _NC * _NS
_RPW = _R // _NW
_CH = 64
_NCHUNK = _RPW // _CH

_BLK = 1024


def _layer2_body(x_ref, h_ref, m_ref, ws_ref, wh_ref, b_ref, o_ref, os_ref):
    x = x_ref[...]
    h = h_ref[...]
    acc = jnp.dot(x, ws_ref[...], preferred_element_type=jnp.float32)
    acc += jnp.dot(h, wh_ref[...], preferred_element_type=jnp.float32)
    o = jnp.maximum(acc + b_ref[...], 0.0) * m_ref[...]
    o_ref[...] = o
    os_ref[...] = o


def _layer_body(x_ref, h_ref, m_ref, ws_ref, wh_ref, b_ref, o_ref):
    x = x_ref[...]
    h = h_ref[...]
    acc = jnp.dot(x, ws_ref[...], preferred_element_type=jnp.float32)
    acc += jnp.dot(h, wh_ref[...], preferred_element_type=jnp.float32)
    o_ref[...] = jnp.maximum(acc + b_ref[...], 0.0) * m_ref[...]


_row_spec = pl.BlockSpec((_BLK, _H), lambda i: (i, 0))
_mask_spec = pl.BlockSpec((_BLK, 1), lambda i: (i, 0))
_w_spec = pl.BlockSpec((_H, _H), lambda i: (0, 0))
_b_spec = pl.BlockSpec((1, _H), lambda i: (0, 0))

_tc_layer1 = pl.pallas_call(
    _layer2_body,
    grid=(_R // _BLK,),
    in_specs=[_row_spec, _row_spec, _mask_spec, _w_spec, _w_spec, _b_spec],
    out_specs=(_row_spec, _row_spec),
    out_shape=(jax.ShapeDtypeStruct((_R, _H), jnp.float32),
               jax.ShapeDtypeStruct((_R, _H), jnp.float32)),
)

_tc_layer2 = pl.pallas_call(
    _layer_body,
    grid=(_R // _BLK,),
    in_specs=[_row_spec, _row_spec, _mask_spec, _w_spec, _w_spec, _b_spec],
    out_specs=_row_spec,
    out_shape=jax.ShapeDtypeStruct((_R, _H), jnp.float32),
)


def _sc_gather_body(heads_hbm, table_hbm, out_hbm, idx_v, buf0, buf1, sem0, sem1):
    wid = lax.axis_index("s") * _NC + lax.axis_index("c")
    base = wid * _RPW
    pltpu.sync_copy(heads_hbm.at[pl.ds(base, _RPW)], idx_v)
    off = (base // _S) * _S
    for j in range(_RPW // _L):
        sl = pl.ds(j * _L, _L)
        idx_v[sl] = idx_v[sl] + off
    bufs, sems = (buf0, buf1), (sem0, sem1)
    cps = []
    for i in range(_NCHUNK):
        cp = pltpu.make_async_copy(
            table_hbm.at[idx_v.at[pl.ds(i * _CH, _CH)]], bufs[i % 2], sems[i % 2])
        cp.start()
        cps.append(cp)
        if i >= 1:
            cps[i - 1].wait()
            pltpu.sync_copy(bufs[(i - 1) % 2],
                            out_hbm.at[pl.ds(base + (i - 1) * _CH, _CH)])
    cps[-1].wait()
    pltpu.sync_copy(bufs[(_NCHUNK - 1) % 2],
                    out_hbm.at[pl.ds(base + (_NCHUNK - 1) * _CH, _CH)])


@functools.cache
def _make_sc_gather():
    return pl.kernel(
        _sc_gather_body,
        out_type=jax.ShapeDtypeStruct((_R, _H), jnp.float32),
        mesh=plsc.VectorSubcoreMesh(core_axis_name="c", subcore_axis_name="s"),
        scratch_types=[
            pltpu.VMEM((_RPW,), jnp.int32),
            pltpu.VMEM((_CH, _H), jnp.float32),
            pltpu.VMEM((_CH, _H), jnp.float32),
            pltpu.SemaphoreType.DMA,
            pltpu.SemaphoreType.DMA,
        ],
    )


def kernel(hidden_states, attention_mask, heads, rels, W_self, W_head, b):
    del rels
    x0 = hidden_states.reshape(_R, _H)
    mask = attention_mask.reshape(_R, 1)
    hflat = heads.reshape(_R).astype(jnp.int32)

    sc_gather = _make_sc_gather()
    h1 = sc_gather(hflat, x0)
    x1, x1s = _tc_layer1(x0, h1, mask, W_self[0], W_head[0], b[0].reshape(1, _H))
    h2 = sc_gather(hflat, x1s)
    x2 = _tc_layer2(x1, h2, mask, W_self[1], W_head[1], b[1].reshape(1, _H))
    return x2.reshape(_B, _S, _H)
